# baseline (speedup 1.0000x reference)
import functools

import jax
import jax.numpy as jnp
from jax import lax
from jax.experimental import pallas as pl
from jax.experimental.pallas import tpu as pltpu
from jax.experimental.pallas import tpu_sc as plsc

_B_TC = 32
_BB = 8
_NSTREAM = 4
_CHUNK = 16
_NBUF = 4
_LANES = 16


def _sc_pooled(x3, idx, b_lo, n_workers):
    b, c, hw = x3.shape
    rows = (b - b_lo) * c
    n_idx = idx.shape[0]
    rows_w = rows // n_workers
    n_chunks = rows_w // _CHUNK
    n_groups = n_chunks // _NBUF
    chunks_per_b = c // _CHUNK
    mesh = plsc.VectorSubcoreMesh(core_axis_name="c", subcore_axis_name="s")

    @functools.partial(
        pl.kernel, mesh=mesh,
        out_type=jax.ShapeDtypeStruct((rows,), jnp.float32),
        scratch_types=[
            *[pltpu.VMEM((_CHUNK, hw), jnp.float32) for _ in range(_NBUF)],
            pltpu.VMEM((rows_w,), jnp.float32),
            pltpu.VMEM((n_idx,), jnp.int32),
            pltpu.VMEM((n_idx * _LANES,), jnp.int32),
            pltpu.VMEM((n_idx * _LANES,), jnp.int32),
            *[pltpu.SemaphoreType.DMA for _ in range(_NBUF)],
        ],
        compiler_params=pltpu.CompilerParams(needs_layout_passes=False),
    )
    def k(x_hbm, idx_hbm, out_hbm, *refs):
        bufs = refs[:_NBUF]
        obuf, idx_v, tbl_r, tbl_c = refs[_NBUF:_NBUF + 4]
        sems = refs[_NBUF + 4:]
        wid = lax.axis_index("s") * 2 + lax.axis_index("c")
        pltpu.sync_copy(idx_hbm, idx_v)
        lane = lax.iota(jnp.int32, _LANES)
        for q in range(n_idx // _LANES):
            v = idx_v[pl.ds(q * _LANES, _LANES)]
            for t in range(_LANES):
                pos = lane * _LANES + (q * _LANES * _LANES + t)
                plsc.store_scatter(tbl_r, [pos], jnp.full((_LANES,), t, jnp.int32))
                plsc.store_scatter(tbl_c, [pos], v)

        def _src(chunk):
            gchunk = wid * n_chunks + chunk
            bi = b_lo + gchunk // chunks_per_b
            c0 = (gchunk % chunks_per_b) * _CHUNK
            return x_hbm.at[bi, pl.ds(c0, _CHUNK), :]

        def start_in(chunk, buf, sem):
            pltpu.make_async_copy(_src(chunk), buf, sem).start()

        def wait_in(chunk, buf, sem):
            pltpu.make_async_copy(_src(chunk), buf, sem).wait()

        def consume(chunk, buf):
            acc = plsc.load_gather(
                buf, [tbl_r[pl.ds(0, _LANES)], tbl_c[pl.ds(0, _LANES)]])
            for s in range(1, n_idx):
                acc = acc + plsc.load_gather(
                    buf, [tbl_r[pl.ds(s * _LANES, _LANES)],
                          tbl_c[pl.ds(s * _LANES, _LANES)]])
            obuf[pl.ds(chunk * _CHUNK, _CHUNK)] = acc

        for j in range(_NBUF):
            start_in(j, bufs[j], sems[j])

        def body(g, carry):
            c0 = g * _NBUF
            for j in range(_NBUF):
                wait_in(c0 + j, bufs[j], sems[j])
                consume(c0 + j, bufs[j])

                @pl.when(c0 + j + _NBUF < n_chunks)
                def _():
                    start_in(c0 + j + _NBUF, bufs[j], sems[j])

            return carry

        lax.fori_loop(0, n_groups, body, 0)
        pltpu.sync_copy(obuf, out_hbm.at[pl.ds(wid * rows_w, rows_w)])

    return k(x3, idx)


def _tc_pool_body(idx_ref, *refs):
    x_refs = refs[:_NSTREAM]
    o_ref = refs[_NSTREAM]
    hw = x_refs[0].shape[-1]
    pos = jax.lax.broadcasted_iota(jnp.int32, (1, hw), 1)
    hits = (idx_ref[...] == pos).astype(jnp.float32)
    wt = jnp.sum(hits, axis=0, keepdims=True)
    o_ref[...] = jnp.concatenate(
        [jnp.sum(xr[...] * wt[None, :, :], axis=2) for xr in x_refs],
        axis=1)


def _tc_pool(x3, idx2, b_hi):
    b, c, hw = x3.shape
    n_idx = idx2.shape[0]
    cs = c // _NSTREAM

    def _mk_spec(k):
        return pl.BlockSpec((_BB, cs, hw), lambda i, k=k: (i, k, 0))

    return pl.pallas_call(
        _tc_pool_body,
        grid=(b_hi // _BB,),
        in_specs=[
            pl.BlockSpec((n_idx, 1), lambda i: (0, 0)),
            *[_mk_spec(k) for k in range(_NSTREAM)],
        ],
        out_specs=pl.BlockSpec((_BB, c), lambda i: (i, 0)),
        out_shape=jax.ShapeDtypeStruct((b_hi, c), jnp.float32),
        compiler_params=pltpu.CompilerParams(
            dimension_semantics=("arbitrary",)),
    )(idx2, *([x3] * _NSTREAM))


def _mm_body(p_ref, w_ref, o_ref, *, inv_n):
    pooled = p_ref[...] * inv_n
    o_ref[...] = lax.dot_general(
        pooled, w_ref[...], (((1,), (1,)), ((), ())),
        preferred_element_type=jnp.float32)


def kernel(x, W, indice):
    b, c, h, w = x.shape
    hw = h * w
    n, _ = W.shape
    x3 = x.reshape(b, c, hw)
    idx = indice.astype(jnp.int32)
    n_idx = idx.shape[0]

    y_sc = _sc_pooled(x3, idx, _B_TC, n_workers=32)
    y_tc = _tc_pool(x3, idx.reshape(-1, 1), _B_TC)
    pooled = jnp.concatenate([y_tc, y_sc.reshape(b - _B_TC, c)], axis=0)

    out = pl.pallas_call(
        functools.partial(_mm_body, inv_n=1.0 / n_idx),
        in_specs=[
            pl.BlockSpec((b, c), lambda: (0, 0)),
            pl.BlockSpec((n, c), lambda: (0, 0)),
        ],
        out_specs=pl.BlockSpec((b, n), lambda: (0, 0)),
        out_shape=jax.ShapeDtypeStruct((b, n), jnp.float32),
    )(pooled, W)
    return out


# speedup vs baseline: 1.0053x; 1.0053x over previous
import functools

import jax
import jax.numpy as jnp
from jax import lax
from jax.experimental import pallas as pl
from jax.experimental.pallas import tpu as pltpu
from jax.experimental.pallas import tpu_sc as plsc

_B_TC = 96
_BB = 8
_NSTREAM = 4
_CHUNK = 16
_NBUF = 4
_LANES = 16


def _sc_pooled(x3, idx, b_lo, n_workers):
    b, c, hw = x3.shape
    rows = (b - b_lo) * c
    n_idx = idx.shape[0]
    rows_w = rows // n_workers
    n_chunks = rows_w // _CHUNK
    n_groups = n_chunks // _NBUF
    chunks_per_b = c // _CHUNK
    mesh = plsc.VectorSubcoreMesh(core_axis_name="c", subcore_axis_name="s")

    @functools.partial(
        pl.kernel, mesh=mesh,
        out_type=jax.ShapeDtypeStruct((rows,), jnp.float32),
        scratch_types=[
            *[pltpu.VMEM((_CHUNK, hw), jnp.float32) for _ in range(_NBUF)],
            pltpu.VMEM((rows_w,), jnp.float32),
            pltpu.VMEM((n_idx,), jnp.int32),
            pltpu.VMEM((n_idx * _LANES,), jnp.int32),
            pltpu.VMEM((n_idx * _LANES,), jnp.int32),
            *[pltpu.SemaphoreType.DMA for _ in range(_NBUF)],
        ],
        compiler_params=pltpu.CompilerParams(needs_layout_passes=False),
    )
    def k(x_hbm, idx_hbm, out_hbm, *refs):
        bufs = refs[:_NBUF]
        obuf, idx_v, tbl_r, tbl_c = refs[_NBUF:_NBUF + 4]
        sems = refs[_NBUF + 4:]
        wid = lax.axis_index("s") * 2 + lax.axis_index("c")
        pltpu.sync_copy(idx_hbm, idx_v)
        lane = lax.iota(jnp.int32, _LANES)
        for q in range(n_idx // _LANES):
            v = idx_v[pl.ds(q * _LANES, _LANES)]
            for t in range(_LANES):
                pos = lane * _LANES + (q * _LANES * _LANES + t)
                plsc.store_scatter(tbl_r, [pos], jnp.full((_LANES,), t, jnp.int32))
                plsc.store_scatter(tbl_c, [pos], v)

        def _src(chunk):
            gchunk = wid * n_chunks + chunk
            bi = b_lo + gchunk // chunks_per_b
            c0 = (gchunk % chunks_per_b) * _CHUNK
            return x_hbm.at[bi, pl.ds(c0, _CHUNK), :]

        def start_in(chunk, buf, sem):
            pltpu.make_async_copy(_src(chunk), buf, sem).start()

        def wait_in(chunk, buf, sem):
            pltpu.make_async_copy(_src(chunk), buf, sem).wait()

        def consume(chunk, buf):
            acc = plsc.load_gather(
                buf, [tbl_r[pl.ds(0, _LANES)], tbl_c[pl.ds(0, _LANES)]])
            for s in range(1, n_idx):
                acc = acc + plsc.load_gather(
                    buf, [tbl_r[pl.ds(s * _LANES, _LANES)],
                          tbl_c[pl.ds(s * _LANES, _LANES)]])
            obuf[pl.ds(chunk * _CHUNK, _CHUNK)] = acc

        for j in range(_NBUF):
            start_in(j, bufs[j], sems[j])

        def body(g, carry):
            c0 = g * _NBUF
            for j in range(_NBUF):
                wait_in(c0 + j, bufs[j], sems[j])
                consume(c0 + j, bufs[j])

                @pl.when(c0 + j + _NBUF < n_chunks)
                def _():
                    start_in(c0 + j + _NBUF, bufs[j], sems[j])

            return carry

        lax.fori_loop(0, n_groups, body, 0)
        pltpu.sync_copy(obuf, out_hbm.at[pl.ds(wid * rows_w, rows_w)])

    return k(x3, idx)


def _tc_pool_body(idx_ref, *refs):
    x_refs = refs[:_NSTREAM]
    o_ref = refs[_NSTREAM]
    hw = x_refs[0].shape[-1]
    pos = jax.lax.broadcasted_iota(jnp.int32, (1, hw), 1)
    hits = (idx_ref[...] == pos).astype(jnp.float32)
    wt = jnp.sum(hits, axis=0, keepdims=True)
    o_ref[...] = jnp.concatenate(
        [jnp.sum(xr[...] * wt[None, :, :], axis=2) for xr in x_refs],
        axis=1)


def _tc_pool(x3, idx2, b_hi):
    b, c, hw = x3.shape
    n_idx = idx2.shape[0]
    cs = c // _NSTREAM

    def _mk_spec(k):
        return pl.BlockSpec((_BB, cs, hw), lambda i, k=k: (i, k, 0))

    return pl.pallas_call(
        _tc_pool_body,
        grid=(b_hi // _BB,),
        in_specs=[
            pl.BlockSpec((n_idx, 1), lambda i: (0, 0)),
            *[_mk_spec(k) for k in range(_NSTREAM)],
        ],
        out_specs=pl.BlockSpec((_BB, c), lambda i: (i, 0)),
        out_shape=jax.ShapeDtypeStruct((b_hi, c), jnp.float32),
        compiler_params=pltpu.CompilerParams(
            dimension_semantics=("arbitrary",)),
    )(idx2, *([x3] * _NSTREAM))


def _mm_body(p_ref, w_ref, o_ref, *, inv_n):
    pooled = p_ref[...] * inv_n
    o_ref[...] = lax.dot_general(
        pooled, w_ref[...], (((1,), (1,)), ((), ())),
        preferred_element_type=jnp.float32)


def kernel(x, W, indice):
    b, c, h, w = x.shape
    hw = h * w
    n, _ = W.shape
    x3 = x.reshape(b, c, hw)
    idx = indice.astype(jnp.int32)
    n_idx = idx.shape[0]

    y_sc = _sc_pooled(x3, idx, _B_TC, n_workers=32)
    y_tc = _tc_pool(x3, idx.reshape(-1, 1), _B_TC)
    pooled = jnp.concatenate([y_tc, y_sc.reshape(b - _B_TC, c)], axis=0)

    out = pl.pallas_call(
        functools.partial(_mm_body, inv_n=1.0 / n_idx),
        in_specs=[
            pl.BlockSpec((b, c), lambda: (0, 0)),
            pl.BlockSpec((n, c), lambda: (0, 0)),
        ],
        out_specs=pl.BlockSpec((b, n), lambda: (0, 0)),
        out_shape=jax.ShapeDtypeStruct((b, n), jnp.float32),
    )(pooled, W)
    return out
